# baseline (speedup 1.0000x reference)
import functools

import jax
import jax.numpy as jnp
import numpy as np
from jax import lax
from jax.experimental import pallas as pl
from jax.experimental.pallas import tpu as pltpu
from jax.experimental.pallas import tpu_sc as plsc

NSAMPLE = 16
RAD2 = np.float32(0.2 ** 2)
LANES = 2048


def _sc_index_body(pos1_hbm, pos2_hbm, fac_hbm, idxc_hbm,
                   qv, p1c, p2c, facv, acc):
    nq = qv.shape[1]
    wid = lax.axis_index("s") * 2 + lax.axis_index("c")
    b = wid // 8
    i0 = (wid % 8) * nq
    pltpu.sync_copy(pos1_hbm.at[b, :, pl.ds(i0, nq)], qv)
    pltpu.sync_copy(pos1_hbm.at[b, :, pl.ds(0, 128)], p1c)
    pltpu.sync_copy(pos2_hbm.at[b, :, pl.ds(0, 128)], p2c)
    pltpu.sync_copy(fac_hbm, facv)

    iota = lax.iota(jnp.int32, 16)
    fvec = facv[pl.ds(0, 16)]
    sent = jnp.full((16,), NSAMPLE, dtype=jnp.int32)

    def rne_bf16(x):
        u = lax.bitcast_convert_type(x, jnp.int32)
        u2 = u + jnp.int32(0x7FFF) + \
            jnp.bitwise_and(lax.shift_right_logical(u, 16), jnp.int32(1))
        return lax.bitcast_convert_type(
            jnp.bitwise_and(u2, jnp.int32(-65536)), jnp.float32)

    c1v = [p1c[c, pl.ds(0, 16)] for c in range(3)]
    c2v = [p2c[c, pl.ds(0, 16)] for c in range(3)]
    pp1 = (c1v[0] * c1v[0] + c1v[1] * c1v[1]) + c1v[2] * c1v[2]
    pp2 = (c2v[0] * c2v[0] + c2v[1] * c2v[1]) + c2v[2] * c2v[2]
    c1v = [rne_bf16(v) for v in c1v]
    c2v = [rne_bf16(v) for v in c2v]

    def ball_pass(qx, qy, qz, qq, cv, pp, col_off, col_lim, rows, val_off):
        rank = jnp.zeros((16,), jnp.int32)
        kmin = sent
        for j in range(16):
            m = (qx * cv[0][j] + qy * cv[1][j]) + qz * cv[2][j]
            d = (-2.0 * m + qq) + pp[j]
            inball = jnp.logical_not(d > RAD2)
            jvec = jnp.full((16,), j, jnp.int32)
            kmin = jnp.minimum(kmin, jnp.where(inball, jvec, sent))
            cols = rank + col_off
            ok = jnp.logical_and(inball, cols < col_lim)
            plsc.store_scatter(acc, [rows * 16 + cols],
                               jnp.full((16,), j + val_off, jnp.int32),
                               mask=ok)
            rank = rank + inball.astype(jnp.int32)
        return rank, kmin

    def qloop(g, carry):
        qb = g * 16
        qx = qv[0, pl.ds(qb, 16)]
        qy = qv[1, pl.ds(qb, 16)]
        qz = qv[2, pl.ds(qb, 16)]
        qq = (qx * qx + qy * qy) + qz * qz
        qx, qy, qz = rne_bf16(qx), rne_bf16(qy), rne_bf16(qz)
        rows = iota + qb
        rank1, kmin1 = ball_pass(qx, qy, qz, qq, c1v, pp1,
                                 jnp.zeros((16,), jnp.int32), fvec, rows, 0)
        rank2, kmin2 = ball_pass(qx, qy, qz, qq, c2v, pp2,
                                 fvec, sent, rows, 17)
        g1 = kmin1
        g2 = kmin2 + 17
        for c in range(16):
            cful = jnp.full((16,), c, jnp.int32)
            is1 = cful < fvec
            fill = jnp.where(is1, rank1 <= cful, rank2 <= cful - fvec)
            val = jnp.where(is1, g1, g2)
            plsc.store_scatter(acc, [rows * 16 + cful], val, mask=fill)
        return carry

    lax.fori_loop(0, nq // 16, qloop, 0)
    pltpu.sync_copy(acc, idxc_hbm.at[b, pl.ds(i0 * 16, nq * 16)])


def _tc_group_body(idx_ref, cf_ref, cp_ref, of_ref, op_ref):
    nb = LANES // 16
    idxl = idx_ref[0]
    iota2 = lax.broadcasted_iota(jnp.int32, (64, LANES), 0)
    oh = (iota2 == idxl).astype(jnp.float32)
    dims = (((1,), (0,)), ((), ()))
    rf = lax.dot_general(cf_ref[0], oh, dims,
                         precision=lax.Precision.HIGHEST,
                         preferred_element_type=jnp.float32)
    rp = lax.dot_general(cp_ref[0], oh, dims,
                         precision=lax.Precision.HIGHEST,
                         preferred_element_type=jnp.float32)
    of_ref[0] = rf.reshape(of_ref.shape[1], nb, 16)
    op_ref[0] = rp.reshape(op_ref.shape[1], nb, 16)


def kernel(pos1, pos2, feats1, feats2, factor):
    B, C, N = feats1.shape
    K = NSAMPLE
    nq = (B * N) // 32
    facv = jnp.full((128,), factor, dtype=jnp.int32)

    sc_idx = pl.kernel(
        _sc_index_body,
        out_type=jax.ShapeDtypeStruct((B, N * K), jnp.int32),
        mesh=plsc.VectorSubcoreMesh(core_axis_name="c", subcore_axis_name="s"),
        compiler_params=pltpu.CompilerParams(needs_layout_passes=False,
                                             use_tc_tiling_on_sc=True),
        scratch_types=[
            pltpu.VMEM((3, nq), jnp.float32),
            pltpu.VMEM((3, 128), jnp.float32),
            pltpu.VMEM((3, 128), jnp.float32),
            pltpu.VMEM((128,), jnp.int32),
            pltpu.VMEM((nq * 16,), jnp.int32),
        ],
    )
    idxc = sc_idx(pos1, pos2, facv)

    zf = jnp.zeros((B, C, 64 - 34), jnp.float32)
    zp = jnp.zeros((B, 3, 64 - 34), jnp.float32)
    cand_f = jnp.concatenate(
        [feats1[:, :, :17], feats2[:, :, :17], zf], axis=2)
    cand_p = jnp.concatenate(
        [pos1[:, :, :17], pos2[:, :, :17], zp], axis=2)

    nblk = (N * K) // LANES
    idx_r = idxc.reshape(B * nblk, 1, LANES)
    out_f, out_p = pl.pallas_call(
        _tc_group_body,
        grid=(B, nblk),
        in_specs=[
            pl.BlockSpec((1, 1, LANES), lambda b, j: (b * nblk + j, 0, 0)),
            pl.BlockSpec((1, C, 64), lambda b, j: (b, 0, 0)),
            pl.BlockSpec((1, 3, 64), lambda b, j: (b, 0, 0)),
        ],
        out_specs=[
            pl.BlockSpec((1, C, LANES // 16, K), lambda b, j: (b, 0, j, 0)),
            pl.BlockSpec((1, 3, LANES // 16, K), lambda b, j: (b, 0, j, 0)),
        ],
        out_shape=[
            jax.ShapeDtypeStruct((B, C, N, K), jnp.float32),
            jax.ShapeDtypeStruct((B, 3, N, K), jnp.float32),
        ],
    )(idx_r, cand_f, cand_p)

    return (out_p, out_f)


# speedup vs baseline: 2.4036x; 2.4036x over previous
import functools

import jax
import jax.numpy as jnp
import numpy as np
from jax import lax
from jax.experimental import pallas as pl
from jax.experimental.pallas import tpu as pltpu
from jax.experimental.pallas import tpu_sc as plsc

NSAMPLE = 16
RAD2 = np.float32(0.2 ** 2)
LANES = 2048


def _sc_index_body(pos1_hbm, pos2_hbm, fac_hbm, idxc_hbm,
                   qv, p1c, p2c, facv, acc):
    nq = qv.shape[1]
    wid = lax.axis_index("s") * 2 + lax.axis_index("c")
    b = wid // 8
    i0 = (wid % 8) * nq
    pltpu.sync_copy(pos1_hbm.at[b, :, pl.ds(i0, nq)], qv)
    pltpu.sync_copy(pos1_hbm.at[b, :, pl.ds(0, 128)], p1c)
    pltpu.sync_copy(pos2_hbm.at[b, :, pl.ds(0, 128)], p2c)
    pltpu.sync_copy(fac_hbm, facv)

    iota = lax.iota(jnp.int32, 16)
    fvec = facv[pl.ds(0, 16)]
    sent = jnp.full((16,), NSAMPLE, dtype=jnp.int32)

    def rne_bf16(x):
        u = lax.bitcast_convert_type(x, jnp.int32)
        u2 = u + jnp.int32(0x7FFF) + \
            jnp.bitwise_and(lax.shift_right_logical(u, 16), jnp.int32(1))
        return lax.bitcast_convert_type(
            jnp.bitwise_and(u2, jnp.int32(-65536)), jnp.float32)

    c1v = [p1c[c, pl.ds(0, 16)] for c in range(3)]
    c2v = [p2c[c, pl.ds(0, 16)] for c in range(3)]
    pp1 = (c1v[0] * c1v[0] + c1v[1] * c1v[1]) + c1v[2] * c1v[2]
    pp2 = (c2v[0] * c2v[0] + c2v[1] * c2v[1]) + c2v[2] * c2v[2]
    c1v = [rne_bf16(v) for v in c1v]
    c2v = [rne_bf16(v) for v in c2v]

    def ball_pass(qx, qy, qz, qq, cv, pp, col_off, col_lim, rows, val_off):
        rank = jnp.zeros((16,), jnp.int32)
        kmin = sent
        for j in range(16):
            m = (qx * cv[0][j] + qy * cv[1][j]) + qz * cv[2][j]
            d = (-2.0 * m + qq) + pp[j]
            inball = jnp.logical_not(d > RAD2)
            jvec = jnp.full((16,), j, jnp.int32)
            kmin = jnp.minimum(kmin, jnp.where(inball, jvec, sent))
            cols = rank + col_off
            ok = jnp.logical_and(inball, cols < col_lim)
            plsc.store_scatter(acc, [cols * nq + rows],
                               jnp.full((16,), j + val_off, jnp.int32),
                               mask=ok)
            rank = rank + inball.astype(jnp.int32)
        return rank, kmin

    def qloop(g, carry):
        qb = g * 16
        qx = qv[0, pl.ds(qb, 16)]
        qy = qv[1, pl.ds(qb, 16)]
        qz = qv[2, pl.ds(qb, 16)]
        qq = (qx * qx + qy * qy) + qz * qz
        qx, qy, qz = rne_bf16(qx), rne_bf16(qy), rne_bf16(qz)
        rows = iota + qb
        rank1, kmin1 = ball_pass(qx, qy, qz, qq, c1v, pp1,
                                 jnp.zeros((16,), jnp.int32), fvec, rows, 0)
        rank2, kmin2 = ball_pass(qx, qy, qz, qq, c2v, pp2,
                                 fvec, sent, rows, 17)
        g1 = kmin1
        g2 = kmin2 + 17
        for c in range(16):
            cful = jnp.full((16,), c, jnp.int32)
            is1 = cful < fvec
            fill = jnp.where(is1, rank1 <= cful, rank2 <= cful - fvec)
            val = jnp.where(is1, g1, g2)
            plsc.store_scatter(acc, [cful * nq + rows], val, mask=fill)
        return carry

    lax.fori_loop(0, nq // 16, qloop, 0)
    for c in range(16):
        pltpu.sync_copy(acc.at[pl.ds(c * nq, nq)],
                        idxc_hbm.at[b, c, pl.ds(i0, nq)])


def _tc_group_body(idx_ref, cf_ref, cp_ref, of_ref, op_ref):
    idxl = idx_ref[0]
    iota2 = lax.broadcasted_iota(jnp.int32, (64, LANES), 0)
    oh = (iota2 == idxl).astype(jnp.float32)
    dims = (((1,), (0,)), ((), ()))
    of_ref[0] = lax.dot_general(cf_ref[0], oh, dims,
                                precision=lax.Precision.HIGHEST,
                                preferred_element_type=jnp.float32)
    op_ref[0] = lax.dot_general(cp_ref[0], oh, dims,
                                precision=lax.Precision.HIGHEST,
                                preferred_element_type=jnp.float32)


def kernel(pos1, pos2, feats1, feats2, factor):
    B, C, N = feats1.shape
    K = NSAMPLE
    nq = (B * N) // 32
    facv = jnp.full((128,), factor, dtype=jnp.int32)

    sc_idx = pl.kernel(
        _sc_index_body,
        out_type=jax.ShapeDtypeStruct((B, K, N), jnp.int32),
        mesh=plsc.VectorSubcoreMesh(core_axis_name="c", subcore_axis_name="s"),
        compiler_params=pltpu.CompilerParams(needs_layout_passes=False,
                                             use_tc_tiling_on_sc=True),
        scratch_types=[
            pltpu.VMEM((3, nq), jnp.float32),
            pltpu.VMEM((3, 128), jnp.float32),
            pltpu.VMEM((3, 128), jnp.float32),
            pltpu.VMEM((128,), jnp.int32),
            pltpu.VMEM((nq * 16,), jnp.int32),
        ],
    )
    idxc = sc_idx(pos1, pos2, facv)

    zf = jnp.zeros((B, C, 64 - 34), jnp.float32)
    zp = jnp.zeros((B, 3, 64 - 34), jnp.float32)
    cand_f = jnp.concatenate(
        [feats1[:, :, :17], feats2[:, :, :17], zf], axis=2)
    cand_p = jnp.concatenate(
        [pos1[:, :, :17], pos2[:, :, :17], zp], axis=2)

    nblk = (N * K) // LANES
    idx_r = idxc.reshape(B * nblk, 1, LANES)
    out_f, out_p = pl.pallas_call(
        _tc_group_body,
        grid=(B, nblk),
        in_specs=[
            pl.BlockSpec((1, 1, LANES), lambda b, j: (b * nblk + j, 0, 0)),
            pl.BlockSpec((1, C, 64), lambda b, j: (b, 0, 0)),
            pl.BlockSpec((1, 3, 64), lambda b, j: (b, 0, 0)),
        ],
        out_specs=[
            pl.BlockSpec((1, C, LANES), lambda b, j: (b, 0, j)),
            pl.BlockSpec((1, 3, LANES), lambda b, j: (b, 0, j)),
        ],
        out_shape=[
            jax.ShapeDtypeStruct((B, C, N * K), jnp.float32),
            jax.ShapeDtypeStruct((B, 3, N * K), jnp.float32),
        ],
    )(idx_r, cand_f, cand_p)

    new_pos1 = jnp.transpose(out_p.reshape(B, 3, K, N), (0, 1, 3, 2))
    new_feats1 = jnp.transpose(out_f.reshape(B, C, K, N), (0, 1, 3, 2))
    return (new_pos1, new_feats1)


# speedup vs baseline: 2.5912x; 1.0780x over previous
import functools

import jax
import jax.numpy as jnp
import numpy as np
from jax import lax
from jax.experimental import pallas as pl
from jax.experimental.pallas import tpu as pltpu
from jax.experimental.pallas import tpu_sc as plsc

NSAMPLE = 16
RAD2 = np.float32(0.2 ** 2)
LANES = 4096


def _sc_index_body(pos1_hbm, pos2_hbm, fac_hbm, idxc_hbm,
                   qv, p1c, p2c, facv, acc):
    nq = qv.shape[1]
    wid = lax.axis_index("s") * 2 + lax.axis_index("c")
    b = wid // 8
    i0 = (wid % 8) * nq
    pltpu.sync_copy(pos1_hbm.at[b, :, pl.ds(i0, nq)], qv)
    pltpu.sync_copy(pos1_hbm.at[b, :, pl.ds(0, 128)], p1c)
    pltpu.sync_copy(pos2_hbm.at[b, :, pl.ds(0, 128)], p2c)
    pltpu.sync_copy(fac_hbm, facv)

    iota = lax.iota(jnp.int32, 16)
    fvec = facv[pl.ds(0, 16)]
    sent = jnp.full((16,), NSAMPLE, dtype=jnp.int32)

    def rne_bf16(x):
        u = lax.bitcast_convert_type(x, jnp.int32)
        u2 = u + jnp.int32(0x7FFF) + \
            jnp.bitwise_and(lax.shift_right_logical(u, 16), jnp.int32(1))
        return lax.bitcast_convert_type(
            jnp.bitwise_and(u2, jnp.int32(-65536)), jnp.float32)

    c1v = [p1c[c, pl.ds(0, 16)] for c in range(3)]
    c2v = [p2c[c, pl.ds(0, 16)] for c in range(3)]
    pp1 = (c1v[0] * c1v[0] + c1v[1] * c1v[1]) + c1v[2] * c1v[2]
    pp2 = (c2v[0] * c2v[0] + c2v[1] * c2v[1]) + c2v[2] * c2v[2]
    c1v = [rne_bf16(v) for v in c1v]
    c2v = [rne_bf16(v) for v in c2v]

    def ball_pass(qx, qy, qz, qq, cv, pp, col_off, col_lim, rows, val_off):
        rank = jnp.zeros((16,), jnp.int32)
        kmin = sent
        for j in range(16):
            m = (qx * cv[0][j] + qy * cv[1][j]) + qz * cv[2][j]
            d = (-2.0 * m + qq) + pp[j]
            inball = jnp.logical_not(d > RAD2)
            jvec = jnp.full((16,), j, jnp.int32)
            kmin = jnp.minimum(kmin, jnp.where(inball, jvec, sent))
            cols = rank + col_off
            ok = jnp.logical_and(inball, cols < col_lim)
            plsc.store_scatter(acc, [cols * nq + rows],
                               jnp.full((16,), j + val_off, jnp.int32),
                               mask=ok)
            rank = rank + inball.astype(jnp.int32)
        return rank, kmin

    def qloop(g, carry):
        qb = g * 16
        qx = qv[0, pl.ds(qb, 16)]
        qy = qv[1, pl.ds(qb, 16)]
        qz = qv[2, pl.ds(qb, 16)]
        qq = (qx * qx + qy * qy) + qz * qz
        qx, qy, qz = rne_bf16(qx), rne_bf16(qy), rne_bf16(qz)
        rows = iota + qb
        rank1, kmin1 = ball_pass(qx, qy, qz, qq, c1v, pp1,
                                 jnp.zeros((16,), jnp.int32), fvec, rows, 0)
        rank2, kmin2 = ball_pass(qx, qy, qz, qq, c2v, pp2,
                                 fvec, sent, rows, 17)
        g1 = kmin1
        g2 = kmin2 + 17
        for c in range(16):
            cful = jnp.full((16,), c, jnp.int32)
            is1 = cful < fvec
            fill = jnp.where(is1, rank1 <= cful, rank2 <= cful - fvec)
            val = jnp.where(is1, g1, g2)
            plsc.store_scatter(acc, [cful * nq + rows], val, mask=fill)
        return carry

    lax.fori_loop(0, nq // 16, qloop, 0)
    for c in range(16):
        pltpu.sync_copy(acc.at[pl.ds(c * nq, nq)],
                        idxc_hbm.at[b, c, pl.ds(i0, nq)])


def _tc_group_body(idx_ref, cf_ref, cp_ref, of_ref, op_ref):
    idxl = idx_ref[0]
    iota2 = lax.broadcasted_iota(jnp.int32, (64, LANES), 0)
    oh = (iota2 == idxl).astype(jnp.float32)
    dims = (((1,), (0,)), ((), ()))
    of_ref[0] = lax.dot_general(cf_ref[0], oh, dims,
                                precision=lax.Precision.HIGHEST,
                                preferred_element_type=jnp.float32)
    op_ref[0] = lax.dot_general(cp_ref[0], oh, dims,
                                precision=lax.Precision.HIGHEST,
                                preferred_element_type=jnp.float32)


def kernel(pos1, pos2, feats1, feats2, factor):
    B, C, N = feats1.shape
    K = NSAMPLE
    nq = (B * N) // 32
    facv = jnp.full((128,), factor, dtype=jnp.int32)

    sc_idx = pl.kernel(
        _sc_index_body,
        out_type=jax.ShapeDtypeStruct((B, K, N), jnp.int32),
        mesh=plsc.VectorSubcoreMesh(core_axis_name="c", subcore_axis_name="s"),
        compiler_params=pltpu.CompilerParams(needs_layout_passes=False,
                                             use_tc_tiling_on_sc=True),
        scratch_types=[
            pltpu.VMEM((3, nq), jnp.float32),
            pltpu.VMEM((3, 128), jnp.float32),
            pltpu.VMEM((3, 128), jnp.float32),
            pltpu.VMEM((128,), jnp.int32),
            pltpu.VMEM((nq * 16,), jnp.int32),
        ],
    )
    idxc = sc_idx(pos1, pos2, facv)

    zf = jnp.zeros((B, C, 64 - 34), jnp.float32)
    zp = jnp.zeros((B, 3, 64 - 34), jnp.float32)
    cand_f = jnp.concatenate(
        [feats1[:, :, :17], feats2[:, :, :17], zf], axis=2)
    cand_p = jnp.concatenate(
        [pos1[:, :, :17], pos2[:, :, :17], zp], axis=2)

    nblk = (N * K) // LANES
    idx_r = idxc.reshape(B * nblk, 1, LANES)
    out_f, out_p = pl.pallas_call(
        _tc_group_body,
        grid=(B, nblk),
        in_specs=[
            pl.BlockSpec((1, 1, LANES), lambda b, j: (b * nblk + j, 0, 0)),
            pl.BlockSpec((1, C, 64), lambda b, j: (b, 0, 0)),
            pl.BlockSpec((1, 3, 64), lambda b, j: (b, 0, 0)),
        ],
        out_specs=[
            pl.BlockSpec((1, C, LANES), lambda b, j: (b, 0, j)),
            pl.BlockSpec((1, 3, LANES), lambda b, j: (b, 0, j)),
        ],
        out_shape=[
            jax.ShapeDtypeStruct((B, C, N * K), jnp.float32),
            jax.ShapeDtypeStruct((B, 3, N * K), jnp.float32),
        ],
    )(idx_r, cand_f, cand_p)

    new_pos1 = jnp.transpose(out_p.reshape(B, 3, K, N), (0, 1, 3, 2))
    new_feats1 = jnp.transpose(out_f.reshape(B, C, K, N), (0, 1, 3, 2))
    return (new_pos1, new_feats1)


# speedup vs baseline: 4.2481x; 1.6394x over previous
import functools

import jax
import jax.numpy as jnp
import numpy as np
from jax import lax
from jax.experimental import pallas as pl
from jax.experimental.pallas import tpu as pltpu
from jax.experimental.pallas import tpu_sc as plsc

NSAMPLE = 16
RAD2 = np.float32(0.2 ** 2)
LANES = 4096


def _sc_index_body(pos1_hbm, pos2_hbm, fac_hbm, idxc_hbm,
                   qv, p1c, p2c, facv, acc):
    nq = qv.shape[1]
    wid = lax.axis_index("s") * 2 + lax.axis_index("c")
    b = wid // 8
    i0 = (wid % 8) * nq
    pltpu.sync_copy(pos1_hbm.at[b, :, pl.ds(i0, nq)], qv)
    pltpu.sync_copy(pos1_hbm.at[b, :, pl.ds(0, 128)], p1c)
    pltpu.sync_copy(pos2_hbm.at[b, :, pl.ds(0, 128)], p2c)
    pltpu.sync_copy(fac_hbm, facv)

    iota = lax.iota(jnp.int32, 16)
    fvec = facv[pl.ds(0, 16)]
    sent = jnp.full((16,), NSAMPLE, dtype=jnp.int32)

    def rne_bf16(x):
        u = lax.bitcast_convert_type(x, jnp.int32)
        u2 = u + jnp.int32(0x7FFF) + \
            jnp.bitwise_and(lax.shift_right_logical(u, 16), jnp.int32(1))
        return lax.bitcast_convert_type(
            jnp.bitwise_and(u2, jnp.int32(-65536)), jnp.float32)

    c1v = [p1c[c, pl.ds(0, 16)] for c in range(3)]
    c2v = [p2c[c, pl.ds(0, 16)] for c in range(3)]
    pp1 = (c1v[0] * c1v[0] + c1v[1] * c1v[1]) + c1v[2] * c1v[2]
    pp2 = (c2v[0] * c2v[0] + c2v[1] * c2v[1]) + c2v[2] * c2v[2]
    c1v = [rne_bf16(v) for v in c1v]
    c2v = [rne_bf16(v) for v in c2v]

    def ball_pass(qx, qy, qz, qq, cv, pp, col_off, col_lim, rows, val_off):
        rank = jnp.zeros((16,), jnp.int32)
        kmin = sent
        for j in range(16):
            m = (qx * cv[0][j] + qy * cv[1][j]) + qz * cv[2][j]
            d = (-2.0 * m + qq) + pp[j]
            inball = jnp.logical_not(d > RAD2)
            jvec = jnp.full((16,), j, jnp.int32)
            kmin = jnp.minimum(kmin, jnp.where(inball, jvec, sent))
            cols = rank + col_off
            ok = jnp.logical_and(inball, cols < col_lim)
            plsc.store_scatter(acc, [cols * nq + rows],
                               jnp.full((16,), j + val_off, jnp.int32),
                               mask=ok)
            rank = rank + inball.astype(jnp.int32)
        return rank, kmin

    def qloop(g, carry):
        qb = g * 16
        qx = qv[0, pl.ds(qb, 16)]
        qy = qv[1, pl.ds(qb, 16)]
        qz = qv[2, pl.ds(qb, 16)]
        qq = (qx * qx + qy * qy) + qz * qz
        qx, qy, qz = rne_bf16(qx), rne_bf16(qy), rne_bf16(qz)
        rows = iota + qb
        rank1, kmin1 = ball_pass(qx, qy, qz, qq, c1v, pp1,
                                 jnp.zeros((16,), jnp.int32), fvec, rows, 0)
        rank2, kmin2 = ball_pass(qx, qy, qz, qq, c2v, pp2,
                                 fvec, sent, rows, 17)
        g1 = kmin1
        g2 = kmin2 + 17
        for c in range(16):
            cful = jnp.full((16,), c, jnp.int32)
            is1 = cful < fvec
            fill = jnp.where(is1, rank1 <= cful, rank2 <= cful - fvec)
            val = jnp.where(is1, g1, g2)
            plsc.store_scatter(acc, [cful * nq + rows], val, mask=fill)
        return carry

    lax.fori_loop(0, nq // 16, qloop, 0)
    for c in range(16):
        pltpu.sync_copy(acc.at[pl.ds(c * nq, nq)],
                        idxc_hbm.at[b, c, pl.ds(i0, nq)])


def _tc_group_body(idx_ref, cf_ref, cp_ref, of_ref, op_ref):
    kk, nbl = idx_ref.shape[1], idx_ref.shape[2]
    lanes = kk * nbl
    idxl = idx_ref[0].reshape(1, lanes)
    iota2 = lax.broadcasted_iota(jnp.int32, (64, lanes), 0)
    oh = (iota2 == idxl).astype(jnp.float32)
    dims = (((1,), (0,)), ((), ()))
    rf = lax.dot_general(cf_ref[0], oh, dims,
                         precision=lax.Precision.HIGHEST,
                         preferred_element_type=jnp.float32)
    rp = lax.dot_general(cp_ref[0], oh, dims,
                         precision=lax.Precision.HIGHEST,
                         preferred_element_type=jnp.float32)
    of_ref[0] = rf.reshape(of_ref.shape[1], kk, nbl)
    op_ref[0] = rp.reshape(op_ref.shape[1], kk, nbl)


def kernel(pos1, pos2, feats1, feats2, factor):
    B, C, N = feats1.shape
    K = NSAMPLE
    nq = (B * N) // 32
    facv = jnp.full((128,), factor, dtype=jnp.int32)

    sc_idx = pl.kernel(
        _sc_index_body,
        out_type=jax.ShapeDtypeStruct((B, K, N), jnp.int32),
        mesh=plsc.VectorSubcoreMesh(core_axis_name="c", subcore_axis_name="s"),
        compiler_params=pltpu.CompilerParams(needs_layout_passes=False,
                                             use_tc_tiling_on_sc=True),
        scratch_types=[
            pltpu.VMEM((3, nq), jnp.float32),
            pltpu.VMEM((3, 128), jnp.float32),
            pltpu.VMEM((3, 128), jnp.float32),
            pltpu.VMEM((128,), jnp.int32),
            pltpu.VMEM((nq * 16,), jnp.int32),
        ],
    )
    idxc = sc_idx(pos1, pos2, facv)

    zf = jnp.zeros((B, C, 64 - 34), jnp.float32)
    zp = jnp.zeros((B, 3, 64 - 34), jnp.float32)
    cand_f = jnp.concatenate(
        [feats1[:, :, :17], feats2[:, :, :17], zf], axis=2)
    cand_p = jnp.concatenate(
        [pos1[:, :, :17], pos2[:, :, :17], zp], axis=2)

    nbl = LANES // K
    nblk = N // nbl
    out_f, out_p = pl.pallas_call(
        _tc_group_body,
        grid=(B, nblk),
        in_specs=[
            pl.BlockSpec((1, K, nbl), lambda b, j: (b, 0, j)),
            pl.BlockSpec((1, C, 64), lambda b, j: (b, 0, 0)),
            pl.BlockSpec((1, 3, 64), lambda b, j: (b, 0, 0)),
        ],
        out_specs=[
            pl.BlockSpec((1, C, K, nbl), lambda b, j: (b, 0, 0, j)),
            pl.BlockSpec((1, 3, K, nbl), lambda b, j: (b, 0, 0, j)),
        ],
        out_shape=[
            jax.ShapeDtypeStruct((B, C, K, N), jnp.float32),
            jax.ShapeDtypeStruct((B, 3, K, N), jnp.float32),
        ],
    )(idxc, cand_f, cand_p)

    new_pos1 = jnp.transpose(out_p, (0, 1, 3, 2))
    new_feats1 = jnp.transpose(out_f, (0, 1, 3, 2))
    return (new_pos1, new_feats1)


# speedup vs baseline: 5.7231x; 1.3472x over previous
import functools

import jax
import jax.numpy as jnp
import numpy as np
from jax import lax
from jax.experimental import pallas as pl
from jax.experimental.pallas import tpu as pltpu
from jax.experimental.pallas import tpu_sc as plsc

NSAMPLE = 16
RAD2 = np.float32(0.2 ** 2)
LANES = 4096


def _sc_index_body(pos1_hbm, pos2_hbm, fac_hbm, idxc_hbm,
                   qv, p1c, p2c, facv, acc):
    nq = qv.shape[1]
    wid = lax.axis_index("s") * 2 + lax.axis_index("c")
    b = wid // 8
    i0 = (wid % 8) * nq
    pltpu.sync_copy(pos1_hbm.at[b, :, pl.ds(i0, nq)], qv)
    pltpu.sync_copy(pos1_hbm.at[b, :, pl.ds(0, 128)], p1c)
    pltpu.sync_copy(pos2_hbm.at[b, :, pl.ds(0, 128)], p2c)
    pltpu.sync_copy(fac_hbm, facv)

    iota = lax.iota(jnp.int32, 16)
    fvec = facv[pl.ds(0, 16)]
    sent = jnp.full((16,), NSAMPLE, dtype=jnp.int32)

    def rne_bf16(x):
        u = lax.bitcast_convert_type(x, jnp.int32)
        u2 = u + jnp.int32(0x7FFF) + \
            jnp.bitwise_and(lax.shift_right_logical(u, 16), jnp.int32(1))
        return lax.bitcast_convert_type(
            jnp.bitwise_and(u2, jnp.int32(-65536)), jnp.float32)

    c1v = [p1c[c, pl.ds(0, 16)] for c in range(3)]
    c2v = [p2c[c, pl.ds(0, 16)] for c in range(3)]
    pp1 = (c1v[0] * c1v[0] + c1v[1] * c1v[1]) + c1v[2] * c1v[2]
    pp2 = (c2v[0] * c2v[0] + c2v[1] * c2v[1]) + c2v[2] * c2v[2]
    c1v = [rne_bf16(v) for v in c1v]
    c2v = [rne_bf16(v) for v in c2v]

    def ball_pass(qx, qy, qz, qq, cv, pp, col_off, col_lim, rows, val_off):
        rank = jnp.zeros((16,), jnp.int32)
        kmin = sent
        for j in range(16):
            m = (qx * cv[0][j] + qy * cv[1][j]) + qz * cv[2][j]
            d = (-2.0 * m + qq) + pp[j]
            inball = jnp.logical_not(d > RAD2)
            jvec = jnp.full((16,), j, jnp.int32)
            kmin = jnp.minimum(kmin, jnp.where(inball, jvec, sent))
            cols = rank + col_off
            ok = jnp.logical_and(inball, cols < col_lim)
            plsc.store_scatter(acc, [cols * nq + rows],
                               jnp.full((16,), j + val_off, jnp.int32),
                               mask=ok)
            rank = rank + inball.astype(jnp.int32)
        return rank, kmin

    def qloop(g, carry):
        qb = g * 16
        qx = qv[0, pl.ds(qb, 16)]
        qy = qv[1, pl.ds(qb, 16)]
        qz = qv[2, pl.ds(qb, 16)]
        qq = (qx * qx + qy * qy) + qz * qz
        qx, qy, qz = rne_bf16(qx), rne_bf16(qy), rne_bf16(qz)
        rows = iota + qb
        rank1, kmin1 = ball_pass(qx, qy, qz, qq, c1v, pp1,
                                 jnp.zeros((16,), jnp.int32), fvec, rows, 0)
        rank2, kmin2 = ball_pass(qx, qy, qz, qq, c2v, pp2,
                                 fvec, sent, rows, 17)
        g1 = kmin1
        g2 = kmin2 + 17
        for c in range(16):
            cful = jnp.full((16,), c, jnp.int32)
            is1 = cful < fvec
            fill = jnp.where(is1, rank1 <= cful, rank2 <= cful - fvec)
            val = jnp.where(is1, g1, g2)
            plsc.store_scatter(acc, [cful * nq + rows], val, mask=fill)
        return carry

    lax.fori_loop(0, nq // 16, qloop, 0)
    for c in range(16):
        pltpu.sync_copy(acc.at[pl.ds(c * nq, nq)],
                        idxc_hbm.at[b, c, pl.ds(i0, nq)])


def _tc_group_body(idx_ref, cfh_ref, cfm_ref, cfl_ref,
                   cph_ref, cpm_ref, cpl_ref, of_ref, op_ref):
    kk, nbl = idx_ref.shape[1], idx_ref.shape[2]
    lanes = kk * nbl
    idxl = idx_ref[0].reshape(1, lanes)
    iota2 = lax.broadcasted_iota(jnp.int32, (64, lanes), 0)
    oh = (iota2 == idxl).astype(jnp.bfloat16)
    dims = (((1,), (0,)), ((), ()))

    def dot3(h_ref, m_ref, l_ref):
        parts = [lax.dot_general(r[0], oh, dims,
                                 preferred_element_type=jnp.float32)
                 for r in (h_ref, m_ref, l_ref)]
        return (parts[0] + parts[1]) + parts[2]

    rf = dot3(cfh_ref, cfm_ref, cfl_ref)
    rp = dot3(cph_ref, cpm_ref, cpl_ref)
    of_ref[0] = rf.reshape(of_ref.shape[1], kk, nbl)
    op_ref[0] = rp.reshape(op_ref.shape[1], kk, nbl)


def kernel(pos1, pos2, feats1, feats2, factor):
    B, C, N = feats1.shape
    K = NSAMPLE
    nq = (B * N) // 32
    facv = jnp.full((128,), factor, dtype=jnp.int32)

    sc_idx = pl.kernel(
        _sc_index_body,
        out_type=jax.ShapeDtypeStruct((B, K, N), jnp.int32),
        mesh=plsc.VectorSubcoreMesh(core_axis_name="c", subcore_axis_name="s"),
        compiler_params=pltpu.CompilerParams(needs_layout_passes=False,
                                             use_tc_tiling_on_sc=True),
        scratch_types=[
            pltpu.VMEM((3, nq), jnp.float32),
            pltpu.VMEM((3, 128), jnp.float32),
            pltpu.VMEM((3, 128), jnp.float32),
            pltpu.VMEM((128,), jnp.int32),
            pltpu.VMEM((nq * 16,), jnp.int32),
        ],
    )
    idxc = sc_idx(pos1, pos2, facv)

    zf = jnp.zeros((B, C, 64 - 34), jnp.float32)
    zp = jnp.zeros((B, 3, 64 - 34), jnp.float32)
    cand_f = jnp.concatenate(
        [feats1[:, :, :17], feats2[:, :, :17], zf], axis=2)
    cand_p = jnp.concatenate(
        [pos1[:, :, :17], pos2[:, :, :17], zp], axis=2)

    def split3(x):
        hi = x.astype(jnp.bfloat16)
        r1 = x - hi.astype(jnp.float32)
        mid = r1.astype(jnp.bfloat16)
        lo = (r1 - mid.astype(jnp.float32)).astype(jnp.bfloat16)
        return hi, mid, lo

    cfh, cfm, cfl = split3(cand_f)
    cph, cpm, cpl = split3(cand_p)

    nbl = LANES // K
    nblk = N // nbl
    out_f, out_p = pl.pallas_call(
        _tc_group_body,
        grid=(B, nblk),
        in_specs=[
            pl.BlockSpec((1, K, nbl), lambda b, j: (b, 0, j)),
            pl.BlockSpec((1, C, 64), lambda b, j: (b, 0, 0)),
            pl.BlockSpec((1, C, 64), lambda b, j: (b, 0, 0)),
            pl.BlockSpec((1, C, 64), lambda b, j: (b, 0, 0)),
            pl.BlockSpec((1, 3, 64), lambda b, j: (b, 0, 0)),
            pl.BlockSpec((1, 3, 64), lambda b, j: (b, 0, 0)),
            pl.BlockSpec((1, 3, 64), lambda b, j: (b, 0, 0)),
        ],
        out_specs=[
            pl.BlockSpec((1, C, K, nbl), lambda b, j: (b, 0, 0, j)),
            pl.BlockSpec((1, 3, K, nbl), lambda b, j: (b, 0, 0, j)),
        ],
        out_shape=[
            jax.ShapeDtypeStruct((B, C, K, N), jnp.float32),
            jax.ShapeDtypeStruct((B, 3, K, N), jnp.float32),
        ],
    )(idxc, cfh, cfm, cfl, cph, cpm, cpl)

    new_pos1 = jnp.transpose(out_p, (0, 1, 3, 2))
    new_feats1 = jnp.transpose(out_f, (0, 1, 3, 2))
    return (new_pos1, new_feats1)
